# baseline (speedup 1.0000x reference)
import functools

import jax
import jax.numpy as jnp
from jax.experimental import pallas as pl
from jax.experimental.pallas import tpu as pltpu

M = 2048
D_MODEL = 1024
D_FF = 2048
E = 8
TOP_K = 2

_NEG = -1e30


def _routing_body(x_ref, wr_ref, gate_ref):
    logits = jax.lax.dot(x_ref[...], wr_ref[...],
                         preferred_element_type=jnp.float32)
    lane = jax.lax.broadcasted_iota(jnp.int32, logits.shape, 1)
    valid = lane < E
    l = jnp.where(valid, logits, _NEG)
    m1 = jnp.max(l, axis=1, keepdims=True)
    i1 = jnp.min(jnp.where(l >= m1, lane, 999), axis=1, keepdims=True)
    l2 = jnp.where(lane == i1, _NEG, l)
    m2 = jnp.max(l2, axis=1, keepdims=True)
    i2 = jnp.min(jnp.where(l2 >= m2, lane, 999), axis=1, keepdims=True)
    w0 = 1.0 / (1.0 + jnp.exp(m2 - m1))
    w1 = 1.0 - w0
    gate = jnp.where(lane == i1, w0, 0.0) + jnp.where(lane == i2, w1, 0.0)
    gate_ref[...] = gate[:, :E]


def _moe_body(x_ref, w_ref, gate_ref, out_ref):
    e = pl.program_id(1)
    part = jax.lax.dot_general(
        x_ref[...], w_ref[0],
        dimension_numbers=(((1,), (1,)), ((), ())),
        preferred_element_type=jnp.float32)
    g = gate_ref[...]
    lane = jax.lax.broadcasted_iota(jnp.int32, g.shape, 1)
    gcol = jnp.sum(jnp.where(lane == e, g, 0.0), axis=1, keepdims=True)
    part = part * gcol

    @pl.when(e == 0)
    def _():
        out_ref[...] = part

    @pl.when(e > 0)
    def _():
        out_ref[...] += part


@functools.partial(jax.jit, static_argnames=())
def kernel(x, W_router, W_experts):
    wr_pad = jnp.pad(W_router, ((0, 0), (0, 128 - E)))
    gate = pl.pallas_call(
        _routing_body,
        out_shape=jax.ShapeDtypeStruct((M, E), jnp.float32),
    )(x, wr_pad)

    x16 = x.astype(jnp.bfloat16)
    w16 = W_experts.astype(jnp.bfloat16)

    BN = 256
    out = pl.pallas_call(
        _moe_body,
        grid=(D_FF // BN, E),
        in_specs=[
            pl.BlockSpec((M, D_MODEL), lambda n, e: (0, 0)),
            pl.BlockSpec((1, BN, D_MODEL), lambda n, e: (e, n, 0)),
            pl.BlockSpec((M, E), lambda n, e: (0, 0)),
        ],
        out_specs=pl.BlockSpec((M, BN), lambda n, e: (0, n)),
        out_shape=jax.ShapeDtypeStruct((M, D_FF), jnp.float32),
    )(x16, w16, gate)
    return out
